# Initial kernel scaffold; baseline (speedup 1.0000x reference)
import jax
import jax.numpy as jnp
from jax.experimental import pallas as pl


def kernel(inp, length, n_batchs):
    raise NotImplementedError("write your pallas kernel here")


# speedup vs baseline: 3.5470x; 3.5470x over previous
import functools

import jax
import jax.numpy as jnp
from jax import lax
from jax.experimental import pallas as pl
from jax.experimental.pallas import tpu as pltpu
from jax.experimental.pallas import tpu_sc as plsc

N_SEG = 64
LANES = 16
NC, NS = 2, 16
NW = NC * NS
ROWS = 16384
COLS = 2048
RPW = ROWS // NW
R_CHUNK = 16
N_CHUNK = RPW // R_CHUNK


def _column_indices(T, lane):
    cols = []
    for v in range(N_SEG // LANES):
        j = lane + v * LANES
        num = (T - 1) * j
        q = num >> 6
        rem = num & 63
        tie_up = (rem == 32) & ((q & 1) == 0)
        inc = jnp.where((rem > 32) | tie_up, 1, 0)
        cols.append(q + inc)
    return cols


def _make_sc_kernel():
    mesh = plsc.VectorSubcoreMesh(core_axis_name="c", subcore_axis_name="s")

    @functools.partial(
        pl.kernel,
        mesh=mesh,
        compiler_params=pltpu.CompilerParams(needs_layout_passes=False),
        out_type=jax.ShapeDtypeStruct((ROWS, N_SEG), jnp.float32),
        scratch_types=[
            pltpu.VMEM((LANES,), jnp.int32),
            pltpu.VMEM((R_CHUNK, COLS), jnp.float32),
            pltpu.VMEM((RPW, N_SEG), jnp.float32),
        ],
    )
    def k(inp_hbm, len_hbm, out_hbm, len_v, rows_v, out_v):
        wid = lax.axis_index("s") * NC + lax.axis_index("c")
        row0 = wid * RPW
        g = wid // 2

        pltpu.sync_copy(len_hbm, len_v)
        lane = lax.iota(jnp.int32, LANES)
        T0 = plsc.load_gather(len_v, [jnp.full((LANES,), g, jnp.int32)])
        T = jnp.where(T0 < 2 * N_SEG, (2 * N_SEG // T0 + 1) * T0, T0)
        cols = _column_indices(T, lane)

        def chunk_body(c, carry):
            pltpu.sync_copy(
                inp_hbm.at[pl.ds(row0 + c * R_CHUNK, R_CHUNK)], rows_v)

            def row_body(r, carry2):
                rsp = jnp.full((LANES,), r, jnp.int32)
                for v in range(N_SEG // LANES):
                    vec = plsc.load_gather(rows_v, [rsp, cols[v]])
                    out_v[c * R_CHUNK + r, pl.ds(v * LANES, LANES)] = vec
                return carry2

            lax.fori_loop(0, R_CHUNK, row_body, 0)
            return carry

        lax.fori_loop(0, N_CHUNK, chunk_body, 0)
        pltpu.sync_copy(out_v, out_hbm.at[pl.ds(row0, RPW)])

    return k


_sc_kernel = _make_sc_kernel()


def kernel(inp, length, n_batchs):
    del n_batchs
    return _sc_kernel(inp, length.astype(jnp.int32))


# speedup vs baseline: 3.9512x; 1.1140x over previous
import functools

import jax
import jax.numpy as jnp
from jax import lax
from jax.experimental import pallas as pl
from jax.experimental.pallas import tpu as pltpu
from jax.experimental.pallas import tpu_sc as plsc

N_SEG = 64
LANES = 16
NC, NS = 2, 16
NW = NC * NS
ROWS = 16384
COLS = 2048
RPW = ROWS // NW
R_CHUNK = 16
N_CHUNK = RPW // R_CHUNK
W_TILE = 512
NT = COLS // W_TILE


def _round_idx(num):
    q = num >> 6
    rem = num & 63
    tie_up = (rem == 32) & ((q & 1) == 0)
    inc = jnp.where((rem > 32) | tie_up, 1, 0)
    return q + inc


def _make_sc_kernel():
    mesh = plsc.VectorSubcoreMesh(core_axis_name="c", subcore_axis_name="s")

    @functools.partial(
        pl.kernel,
        mesh=mesh,
        compiler_params=pltpu.CompilerParams(needs_layout_passes=False),
        out_type=jax.ShapeDtypeStruct((ROWS * N_SEG,), jnp.float32),
        scratch_types=[
            pltpu.VMEM((LANES,), jnp.int32),
            pltpu.VMEM((NT, R_CHUNK, W_TILE), jnp.float32),
            pltpu.VMEM((NT, R_CHUNK, W_TILE), jnp.float32),
            pltpu.VMEM((RPW * N_SEG,), jnp.float32),
            pltpu.SemaphoreType.DMA,
            pltpu.SemaphoreType.DMA,
        ],
    )
    def k(inp_hbm, len_hbm, out_hbm, len_v, buf_a, buf_b, out_v, sem_a, sem_b):
        wid = lax.axis_index("s") * NC + lax.axis_index("c")
        row0 = wid * RPW
        g = wid // 2

        pltpu.sync_copy(len_hbm, len_v)
        lane = lax.iota(jnp.int32, LANES)
        T0 = jnp.max(jnp.where(lane == g, len_v[...], 0))
        T = jnp.where(T0 < 2 * N_SEG, (2 * N_SEG // T0 + 1) * T0, T0)
        idx_max = _round_idx((T - 1) * (N_SEG - 1))
        ntiles = (idx_max >> 9) + 1

        cols = [_round_idx((T - 1) * (lane + v * LANES))
                for v in range(N_SEG // LANES)]
        tile_of = [c >> 9 for c in cols]
        within = [c & (W_TILE - 1) for c in cols]

        def start(c, buf, sem):
            def t_body(t, carry):
                pltpu.async_copy(
                    inp_hbm.at[pl.ds(row0 + c * R_CHUNK, R_CHUNK),
                               pl.ds(t * W_TILE, W_TILE)],
                    buf.at[t], sem)
                return carry
            lax.fori_loop(0, ntiles, t_body, 0)

        def wait(buf, sem):
            def t_body(t, carry):
                pltpu.make_async_copy(
                    inp_hbm.at[pl.ds(row0, R_CHUNK), pl.ds(0, W_TILE)],
                    buf.at[0], sem).wait()
                return carry
            lax.fori_loop(0, ntiles, t_body, 0)

        def compute(c, buf):
            def row_body(r, carry):
                rsp = jnp.full((LANES,), r, jnp.int32)
                for v in range(N_SEG // LANES):
                    vec = plsc.load_gather(buf, [tile_of[v], rsp, within[v]])
                    out_v[pl.ds((c * R_CHUNK + r) * N_SEG + v * LANES,
                                LANES)] = vec
                return carry
            lax.fori_loop(0, R_CHUNK, row_body, 0)

        NH = N_CHUNK // 2
        start(0, buf_a, sem_a)

        def body(i, carry):
            c0 = 2 * i
            start(c0 + 1, buf_b, sem_b)
            wait(buf_a, sem_a)
            compute(c0, buf_a)

            @pl.when(i + 1 < NH)
            def _prefetch():
                start(c0 + 2, buf_a, sem_a)

            wait(buf_b, sem_b)
            compute(c0 + 1, buf_b)
            return carry

        lax.fori_loop(0, NH, body, 0)
        pltpu.sync_copy(out_v, out_hbm.at[pl.ds(row0 * N_SEG, RPW * N_SEG)])

    return k


_sc_kernel = _make_sc_kernel()


def kernel(inp, length, n_batchs):
    del n_batchs
    return _sc_kernel(inp, length.astype(jnp.int32)).reshape(ROWS, N_SEG)


# speedup vs baseline: 3.9887x; 1.0095x over previous
import functools

import jax
import jax.numpy as jnp
from jax import lax
from jax.experimental import pallas as pl
from jax.experimental.pallas import tpu as pltpu
from jax.experimental.pallas import tpu_sc as plsc

N_SEG = 64
LANES = 16
NC, NS = 2, 16
NW = NC * NS
ROWS = 16384
COLS = 2048
RPW = ROWS // NW
R_CHUNK = 16
N_CHUNK = RPW // R_CHUNK
W_TILE = 512
NT = COLS // W_TILE


def _round_idx(num):
    q = num >> 6
    rem = num & 63
    tie_up = (rem == 32) & ((q & 1) == 0)
    inc = jnp.where((rem > 32) | tie_up, 1, 0)
    return q + inc


def _make_sc_kernel():
    mesh = plsc.VectorSubcoreMesh(core_axis_name="c", subcore_axis_name="s")

    @functools.partial(
        pl.kernel,
        mesh=mesh,
        compiler_params=pltpu.CompilerParams(needs_layout_passes=False),
        out_type=jax.ShapeDtypeStruct((ROWS * N_SEG,), jnp.float32),
        scratch_types=[
            pltpu.VMEM((LANES,), jnp.int32),
            pltpu.VMEM((NT * R_CHUNK, W_TILE), jnp.float32),
            pltpu.VMEM((NT * R_CHUNK, W_TILE), jnp.float32),
            pltpu.VMEM((RPW * N_SEG,), jnp.float32),
            pltpu.SemaphoreType.DMA,
            pltpu.SemaphoreType.DMA,
        ],
    )
    def k(inp_hbm, len_hbm, out_hbm, len_v, buf_a, buf_b, out_v, sem_a, sem_b):
        wid = lax.axis_index("s") * NC + lax.axis_index("c")
        row0 = wid * RPW
        g = wid // 2

        pltpu.sync_copy(len_hbm, len_v)
        lane = lax.iota(jnp.int32, LANES)
        T0 = jnp.max(jnp.where(lane == g, len_v[...], 0))
        T = jnp.where(T0 < 2 * N_SEG, (2 * N_SEG // T0 + 1) * T0, T0)
        idx_max = _round_idx((T - 1) * (N_SEG - 1))
        ntiles = (idx_max >> 9) + 1

        cols = [_round_idx((T - 1) * (lane + v * LANES))
                for v in range(N_SEG // LANES)]
        tile_row0 = [(c >> 9) * R_CHUNK for c in cols]
        within = [c & (W_TILE - 1) for c in cols]

        def start(c, buf, sem):
            def t_body(t, carry):
                pltpu.async_copy(
                    inp_hbm.at[pl.ds(row0 + c * R_CHUNK, R_CHUNK),
                               pl.ds(t * W_TILE, W_TILE)],
                    buf.at[pl.ds(t * R_CHUNK, R_CHUNK)], sem)
                return carry
            lax.fori_loop(0, ntiles, t_body, 0)

        def wait(buf, sem):
            def t_body(t, carry):
                pltpu.make_async_copy(
                    inp_hbm.at[pl.ds(row0, R_CHUNK), pl.ds(0, W_TILE)],
                    buf.at[pl.ds(0, R_CHUNK)], sem).wait()
                return carry
            lax.fori_loop(0, ntiles, t_body, 0)

        def compute(c, buf):
            base = c * (R_CHUNK * N_SEG)
            for r in range(R_CHUNK):
                for v in range(N_SEG // LANES):
                    vec = plsc.load_gather(buf, [tile_row0[v] + r, within[v]])
                    out_v[pl.ds(base + r * N_SEG + v * LANES, LANES)] = vec

        NH = N_CHUNK // 2
        start(0, buf_a, sem_a)

        def body(i, carry):
            c0 = 2 * i
            start(c0 + 1, buf_b, sem_b)
            wait(buf_a, sem_a)
            compute(c0, buf_a)

            @pl.when(i + 1 < NH)
            def _prefetch():
                start(c0 + 2, buf_a, sem_a)

            wait(buf_b, sem_b)
            compute(c0 + 1, buf_b)
            return carry

        lax.fori_loop(0, NH, body, 0)
        pltpu.sync_copy(out_v, out_hbm.at[pl.ds(row0 * N_SEG, RPW * N_SEG)])

    return k


_sc_kernel = _make_sc_kernel()


def kernel(inp, length, n_batchs):
    del n_batchs
    return _sc_kernel(inp, length.astype(jnp.int32)).reshape(ROWS, N_SEG)


# speedup vs baseline: 4.6265x; 1.1599x over previous
import functools

import jax
import jax.numpy as jnp
from jax import lax
from jax.experimental import pallas as pl
from jax.experimental.pallas import tpu as pltpu
from jax.experimental.pallas import tpu_sc as plsc

N_SEG = 64
LANES = 16
NC, NS = 2, 16
NW = NC * NS
ROWS = 16384
COLS = 2048
RPW = ROWS // NW
R_CHUNK = 16
N_CHUNK = RPW // R_CHUNK
W_TILE = 512
NT = COLS // W_TILE


def _round_idx(num):
    q = num >> 6
    rem = num & 63
    tie_up = (rem == 32) & ((q & 1) == 0)
    inc = jnp.where((rem > 32) | tie_up, 1, 0)
    return q + inc


def _make_sc_kernel():
    mesh = plsc.VectorSubcoreMesh(core_axis_name="c", subcore_axis_name="s")

    @functools.partial(
        pl.kernel,
        mesh=mesh,
        compiler_params=pltpu.CompilerParams(needs_layout_passes=False),
        out_type=jax.ShapeDtypeStruct((ROWS, N_SEG), jnp.float32),
        scratch_types=[
            pltpu.VMEM((LANES,), jnp.int32),
            pltpu.VMEM((NT * R_CHUNK, W_TILE), jnp.float32),
            pltpu.VMEM((NT * R_CHUNK, W_TILE), jnp.float32),
            pltpu.VMEM((NT * R_CHUNK, W_TILE), jnp.float32),
            pltpu.VMEM((R_CHUNK, N_SEG), jnp.float32),
            pltpu.VMEM((R_CHUNK, N_SEG), jnp.float32),
            pltpu.VMEM((R_CHUNK, N_SEG), jnp.float32),
            pltpu.SemaphoreType.DMA,
            pltpu.SemaphoreType.DMA,
            pltpu.SemaphoreType.DMA,
            pltpu.SemaphoreType.DMA,
            pltpu.SemaphoreType.DMA,
            pltpu.SemaphoreType.DMA,
        ],
    )
    def k(inp_hbm, len_hbm, out_hbm, len_v,
          in0, in1, in2, ob0, ob1, ob2,
          is0, is1, is2, os0, os1, os2):
        wid = lax.axis_index("s") * NC + lax.axis_index("c")
        row0 = wid * RPW
        g = wid // 2

        pltpu.sync_copy(len_hbm, len_v)
        lane = lax.iota(jnp.int32, LANES)
        T0 = jnp.max(jnp.where(lane == g, len_v[...], 0))
        T = jnp.where(T0 < 2 * N_SEG, (2 * N_SEG // T0 + 1) * T0, T0)
        idx_max = _round_idx((T - 1) * (N_SEG - 1))
        ntiles = (idx_max >> 9) + 1

        cols = [_round_idx((T - 1) * (lane + v * LANES))
                for v in range(N_SEG // LANES)]
        tile_row0 = [(c >> 9) * R_CHUNK for c in cols]
        within = [c & (W_TILE - 1) for c in cols]

        ins = [in0, in1, in2]
        obs = [ob0, ob1, ob2]
        isems = [is0, is1, is2]
        osems = [os0, os1, os2]

        def start_in(c, b):
            def t_body(t, carry):
                pltpu.async_copy(
                    inp_hbm.at[pl.ds(row0 + c * R_CHUNK, R_CHUNK),
                               pl.ds(t * W_TILE, W_TILE)],
                    ins[b].at[pl.ds(t * R_CHUNK, R_CHUNK)], isems[b])
                return carry
            lax.fori_loop(0, ntiles, t_body, 0)

        def wait_in(b):
            def t_body(t, carry):
                pltpu.make_async_copy(
                    inp_hbm.at[pl.ds(row0, R_CHUNK), pl.ds(0, W_TILE)],
                    ins[b].at[pl.ds(0, R_CHUNK)], isems[b]).wait()
                return carry
            lax.fori_loop(0, ntiles, t_body, 0)

        def start_out(c, b):
            pltpu.async_copy(
                obs[b], out_hbm.at[pl.ds(row0 + c * R_CHUNK, R_CHUNK)],
                osems[b])

        def wait_out(b):
            pltpu.make_async_copy(
                obs[b], out_hbm.at[pl.ds(row0, R_CHUNK)], osems[b]).wait()

        def compute(c, b):
            buf, ob = ins[b], obs[b]
            for r in range(R_CHUNK):
                for v in range(N_SEG // LANES):
                    vec = plsc.load_gather(buf, [tile_row0[v] + r, within[v]])
                    ob[r, pl.ds(v * LANES, LANES)] = vec

        start_in(0, 0)
        start_in(1, 1)

        def body(i, carry):
            c = 3 * i
            start_in(c + 2, 2)

            wait_in(0)
            @pl.when(i > 0)
            def _w0():
                wait_out(0)
            compute(c, 0)
            start_out(c, 0)
            start_in(c + 3, 0)

            wait_in(1)
            @pl.when(i > 0)
            def _w1():
                wait_out(1)
            compute(c + 1, 1)
            start_out(c + 1, 1)
            start_in(c + 4, 1)

            wait_in(2)
            @pl.when(i > 0)
            def _w2():
                wait_out(2)
            compute(c + 2, 2)
            start_out(c + 2, 2)
            return carry

        NI = N_CHUNK // 3
        lax.fori_loop(0, NI, body, 0)

        wait_in(0)
        wait_out(0)
        compute(N_CHUNK - 2, 0)
        start_out(N_CHUNK - 2, 0)

        wait_in(1)
        wait_out(1)
        compute(N_CHUNK - 1, 1)
        start_out(N_CHUNK - 1, 1)

        wait_out(2)
        wait_out(0)
        wait_out(1)

    return k


_sc_kernel = _make_sc_kernel()


def kernel(inp, length, n_batchs):
    del n_batchs
    return _sc_kernel(inp, length.astype(jnp.int32))
